# baseline (speedup 1.0000x reference)
import functools

import jax
import jax.numpy as jnp
from jax import lax
from jax.experimental import pallas as pl
from jax.experimental.pallas import tpu as pltpu
from jax.experimental.pallas import tpu_sc as plsc

N_E = 8192
E_DIM = 64
BETA = 0.25
ROWS = 512


def _dist_body(zf_ref, emb_ref, esq_ref, iota_ref, idx_ref):
    zb = zf_ref[...]
    zsq = jnp.sum(zb * zb, axis=1, keepdims=True)
    emb = emb_ref[...]
    mm = lax.dot_general(
        zb, emb, (((1,), (1,)), ((), ())),
        preferred_element_type=jnp.float32)
    d = (zsq + esq_ref[...]) - 2.0 * mm
    mval = jnp.min(d, axis=1, keepdims=True)
    idxf = jnp.min(jnp.where(d == mval, iota_ref[...], jnp.float32(1e9)),
                   axis=1)
    idx_ref[...] = idxf.astype(jnp.int32)


def _argmin_indices(zf, esq, embedding):
    n = zf.shape[0]
    nt = n // ROWS
    r128 = ROWS // 128
    iota = jnp.arange(N_E, dtype=jnp.float32).reshape(1, N_E)
    idx = pl.pallas_call(
        _dist_body,
        grid=(nt,),
        in_specs=[
            pl.BlockSpec((ROWS, E_DIM), lambda i: (i, 0)),
            pl.BlockSpec((N_E, E_DIM), lambda i: (0, 0)),
            pl.BlockSpec((1, N_E), lambda i: (0, 0)),
            pl.BlockSpec((1, N_E), lambda i: (0, 0)),
        ],
        out_specs=pl.BlockSpec((ROWS,), lambda i: (i,)),
        out_shape=jax.ShapeDtypeStruct((n,), jnp.int32),
    )(zf, embedding, esq, iota)
    return idx


@functools.lru_cache(maxsize=None)
def _make_sc_gather(B, NC, NS):
    NW = NC * NS
    bw = B // NW
    kc = bw // 128
    mesh = plsc.VectorSubcoreMesh(core_axis_name="c", subcore_axis_name="s")

    @functools.partial(
        pl.kernel,
        out_type=[
            jax.ShapeDtypeStruct((B, E_DIM), jnp.float32),
            jax.ShapeDtypeStruct((NW, 16), jnp.float32),
        ],
        mesh=mesh,
        compiler_params=pltpu.CompilerParams(use_tc_tiling_on_sc=False),
        scratch_types=[
            pltpu.VMEM((bw,), jnp.int32),
            pltpu.VMEM((bw, E_DIM), jnp.float32),
            pltpu.VMEM((bw, E_DIM), jnp.float32),
            pltpu.VMEM((16,), jnp.float32),
            pltpu.SemaphoreType.DMA,
        ],
    )
    def gather_k(emb_hbm, idx_hbm, zf_hbm, qt_hbm, part_hbm,
                 idx_v, rows_v, zf_v, acc_v, sem):
        wid = lax.axis_index("s") * NC + lax.axis_index("c")
        base = wid * bw
        pltpu.sync_copy(idx_hbm.at[pl.ds(base, bw)], idx_v)
        cps = [pltpu.async_copy(emb_hbm.at[idx_v.at[pl.ds(k * 128, 128)]],
                                rows_v.at[pl.ds(k * 128, 128)], sem)
               for k in range(kc)]
        pltpu.sync_copy(zf_hbm.at[pl.ds(base, bw)], zf_v)
        for cp in cps:
            cp.wait()

        def body(j, acc):
            for c in range(E_DIM // 16):
                zq = rows_v[j, pl.ds(c * 16, 16)]
                zb = zf_v[j, pl.ds(c * 16, 16)]
                dq = zq - zb
                rows_v[j, pl.ds(c * 16, 16)] = zb + dq
                acc = acc + dq * dq
            return acc

        acc = lax.fori_loop(0, bw, body, jnp.zeros((16,), jnp.float32))
        acc_v[...] = acc
        pltpu.sync_copy(rows_v, qt_hbm.at[pl.ds(base, bw)])
        pltpu.sync_copy(acc_v, part_hbm.at[wid])

    return gather_k


def kernel(z, embedding):
    b, c, h, w = z.shape
    n = b * h * w
    zf = jnp.transpose(z, (0, 2, 3, 1)).reshape(-1, c)
    esq = jnp.sum(embedding ** 2, axis=1).reshape(1, -1)
    idx2 = _argmin_indices(zf, esq, embedding)
    info = plsc.get_sparse_core_info()
    gather_k = _make_sc_gather(n, info.num_cores, info.num_subcores)
    qt, part = gather_k(embedding, idx2, zf)
    quantized = jnp.transpose(qt.reshape(b, h, w, c), (0, 3, 1, 2))
    m = jnp.sum(part) / (n * c)
    loss = m + BETA * m
    return quantized, loss, idx2


# speedup vs baseline: 1.0372x; 1.0372x over previous
import functools

import jax
import jax.numpy as jnp
from jax import lax
from jax.experimental import pallas as pl
from jax.experimental.pallas import tpu as pltpu
from jax.experimental.pallas import tpu_sc as plsc

N_E = 8192
E_DIM = 64
BETA = 0.25
ROWS = 512


def _dist_body(zf_ref, emb_ref, esq_ref, iota_ref, idx_ref):
    zb = zf_ref[...]
    zsq = jnp.sum(zb * zb, axis=1, keepdims=True)
    emb = emb_ref[...]
    mm = lax.dot_general(
        zb, emb, (((1,), (1,)), ((), ())),
        preferred_element_type=jnp.float32)
    d = (zsq + esq_ref[...]) - 2.0 * mm
    mval = jnp.min(d, axis=1, keepdims=True)
    idxf = jnp.min(jnp.where(d == mval, iota_ref[...], jnp.float32(1e9)),
                   axis=1)
    idx = idxf.astype(jnp.int32)
    idx_ref[...] = idx.reshape(1, ROWS // 128, 128)


def _argmin_indices(zf, esq, embedding):
    n = zf.shape[0]
    nt = n // ROWS
    r128 = ROWS // 128
    iota = jnp.arange(N_E, dtype=jnp.float32).reshape(1, N_E)
    idx = pl.pallas_call(
        _dist_body,
        grid=(nt,),
        in_specs=[
            pl.BlockSpec((ROWS, E_DIM), lambda i: (i, 0)),
            pl.BlockSpec((N_E, E_DIM), lambda i: (0, 0)),
            pl.BlockSpec((1, N_E), lambda i: (0, 0)),
            pl.BlockSpec((1, N_E), lambda i: (0, 0)),
        ],
        out_specs=pl.BlockSpec((1, r128, 128), lambda i: (i, 0, 0)),
        out_shape=jax.ShapeDtypeStruct((nt, r128, 128), jnp.int32),
    )(zf, embedding, esq, iota)
    return idx


@functools.lru_cache(maxsize=None)
def _make_sc_gather(B, NC, NS):
    NW = NC * NS
    bw = B // NW
    kc = bw // 128
    mesh = plsc.VectorSubcoreMesh(core_axis_name="c", subcore_axis_name="s")

    @functools.partial(
        pl.kernel,
        out_type=[
            jax.ShapeDtypeStruct((B, E_DIM), jnp.float32),
            jax.ShapeDtypeStruct((NW, 16), jnp.float32),
        ],
        mesh=mesh,
        compiler_params=pltpu.CompilerParams(use_tc_tiling_on_sc=False),
        scratch_types=[
            pltpu.VMEM((kc, 128), jnp.int32),
            pltpu.VMEM((bw, E_DIM), jnp.float32),
            pltpu.VMEM((bw, E_DIM), jnp.float32),
            pltpu.VMEM((16,), jnp.float32),
            pltpu.SemaphoreType.DMA,
        ],
    )
    def gather_k(emb_hbm, idx_hbm, zf_hbm, qt_hbm, part_hbm,
                 idx_v, rows_v, zf_v, acc_v, sem):
        wid = lax.axis_index("s") * NC + lax.axis_index("c")
        base = wid * bw
        pltpu.sync_copy(idx_hbm.at[wid], idx_v)
        cps = [pltpu.async_copy(emb_hbm.at[idx_v.at[k]],
                                rows_v.at[pl.ds(k * 128, 128)], sem)
               for k in range(kc)]
        pltpu.sync_copy(zf_hbm.at[pl.ds(base, bw)], zf_v)
        for cp in cps:
            cp.wait()

        def body(j, acc):
            for c in range(E_DIM // 16):
                zq = rows_v[j, pl.ds(c * 16, 16)]
                zb = zf_v[j, pl.ds(c * 16, 16)]
                dq = zq - zb
                rows_v[j, pl.ds(c * 16, 16)] = zb + dq
                acc = acc + dq * dq
            return acc

        acc = lax.fori_loop(0, bw, body, jnp.zeros((16,), jnp.float32))
        acc_v[...] = acc
        pltpu.sync_copy(rows_v, qt_hbm.at[pl.ds(base, bw)])
        pltpu.sync_copy(acc_v, part_hbm.at[wid])

    return gather_k


def kernel(z, embedding):
    b, c, h, w = z.shape
    n = b * h * w
    zf = jnp.transpose(z, (0, 2, 3, 1)).reshape(-1, c)
    esq = jnp.sum(embedding ** 2, axis=1).reshape(1, -1)
    idx2 = _argmin_indices(zf, esq, embedding)
    info = plsc.get_sparse_core_info()
    gather_k = _make_sc_gather(n, info.num_cores, info.num_subcores)
    qt, part = gather_k(embedding, idx2, zf)
    quantized = jnp.transpose(qt.reshape(b, h, w, c), (0, 3, 1, 2))
    m = jnp.sum(part) / (n * c)
    loss = m + BETA * m
    min_indices = idx2.reshape(-1)
    return quantized, loss, min_indices
